# Initial kernel scaffold; baseline (speedup 1.0000x reference)
import jax
import jax.numpy as jnp
from jax.experimental import pallas as pl


def kernel(x, edge_index, batch, W1, b1, W2, b2, Wl, bl):
    raise NotImplementedError("write your pallas kernel here")


# speedup vs baseline: 40.5357x; 40.5357x over previous
import functools

import jax
import jax.numpy as jnp
from jax import lax
from jax.experimental import pallas as pl
from jax.experimental.pallas import tpu as pltpu
from jax.experimental.pallas import tpu_sc as plsc

N = 100000
E = 1600000
G = 64
D_IN, D_HID, D_OUT = 32, 16, 5

NPAD = 100352
RPT = NPAD // 16
C = 128
NCHUNK = 12800
E_PAD = NCHUNK * C
CW = NCHUNK // 32
KB = 10
NBLK = CW // KB

R = 5000
NB = N // R

_MESH = plsc.VectorSubcoreMesh(core_axis_name="c", subcore_axis_name="s")
_SC_PARAMS = pltpu.CompilerParams(use_tc_tiling_on_sc=False)


def _worker(c, s):
    return s * 2 + c


@functools.partial(
    pl.kernel,
    out_type=jax.ShapeDtypeStruct((2, NPAD, 1), jnp.float32),
    mesh=_MESH,
    scratch_types=[
        pltpu.VMEM((KB, C), jnp.int32),
        pltpu.VMEM((C, 1), jnp.float32),
        pltpu.VMEM_SHARED((NPAD, 1), jnp.float32),
        pltpu.SemaphoreType.DMA,
    ],
    compiler_params=_SC_PARAMS,
)
def _sc_degree(dst_hbm, z1_hbm, ones_hbm, parts_hbm, dstv, onesv, deg, sem):
    c = lax.axis_index("c")
    s = lax.axis_index("s")
    w = _worker(c, s)
    r0 = s * RPT
    pltpu.sync_copy(z1_hbm.at[pl.ds(r0, RPT)], deg.at[pl.ds(r0, RPT)])
    pltpu.sync_copy(ones_hbm, onesv)
    plsc.subcore_barrier()
    c0 = w * CW

    def block(b, carry):
        base = c0 + b * KB
        pltpu.sync_copy(dst_hbm.at[pl.ds(base, KB)], dstv)
        ds = [pltpu.async_copy(onesv, deg.at[dstv.at[j]], sem, add=True)
              for j in range(KB)]
        for d in ds:
            d.wait()
        return carry

    lax.fori_loop(0, NBLK, block, 0)
    plsc.subcore_barrier()
    pltpu.sync_copy(deg.at[pl.ds(r0, RPT)], parts_hbm.at[c].at[pl.ds(r0, RPT)])


@functools.partial(
    pl.kernel,
    out_type=jax.ShapeDtypeStruct((2, NPAD, D_HID), jnp.float32),
    mesh=_MESH,
    scratch_types=[
        pltpu.VMEM((KB, C), jnp.int32),
        pltpu.VMEM((KB, C), jnp.int32),
        pltpu.VMEM((KB, C, D_HID), jnp.float32),
        pltpu.VMEM_SHARED((NPAD, D_HID), jnp.float32),
        pltpu.SemaphoreType.DMA,
        pltpu.SemaphoreType.DMA,
    ],
    compiler_params=_SC_PARAMS,
)
def _sc_scatter(y_hbm, src_hbm, dst_hbm, z16_hbm, parts_hbm,
                srcv, dstv, rows, acc, semg, sems):
    c = lax.axis_index("c")
    s = lax.axis_index("s")
    w = _worker(c, s)
    r0 = s * RPT
    pltpu.sync_copy(z16_hbm.at[pl.ds(r0, RPT)], acc.at[pl.ds(r0, RPT)])
    plsc.subcore_barrier()
    c0 = w * CW

    def block(b, carry):
        base = c0 + b * KB
        pltpu.sync_copy(src_hbm.at[pl.ds(base, KB)], srcv)
        pltpu.sync_copy(dst_hbm.at[pl.ds(base, KB)], dstv)
        gs = [pltpu.async_copy(y_hbm.at[srcv.at[j]], rows.at[j], semg)
              for j in range(KB)]
        for d in gs:
            d.wait()
        ss = [pltpu.async_copy(rows.at[j], acc.at[dstv.at[j]], sems, add=True)
              for j in range(KB)]
        for d in ss:
            d.wait()
        return carry

    lax.fori_loop(0, NBLK, block, 0)
    plsc.subcore_barrier()
    pltpu.sync_copy(acc.at[pl.ds(r0, RPT)], parts_hbm.at[c].at[pl.ds(r0, RPT)])


def _tc_matmul_body(x_ref, w_ref, o_ref):
    o_ref[...] = jnp.dot(x_ref[...], w_ref[...],
                         preferred_element_type=jnp.float32)


def _tc_matmul(x, w):
    return pl.pallas_call(
        _tc_matmul_body,
        grid=(NB,),
        in_specs=[pl.BlockSpec((R, D_IN), lambda i: (i, 0)),
                  pl.BlockSpec((D_IN, D_HID), lambda i: (0, 0))],
        out_specs=pl.BlockSpec((R, D_HID), lambda i: (i, 0)),
        out_shape=jax.ShapeDtypeStruct((N, D_HID), jnp.float32),
    )(x, w)


def _tc_prep_body(dp_ref, u_ref, y_ref, dinv_ref):
    deg = dp_ref[0] + dp_ref[1] + 1.0
    dinv = lax.rsqrt(deg)
    y_ref[...] = u_ref[...] * dinv
    dinv_ref[...] = jnp.broadcast_to(dinv, (R, D_HID))


def _tc_prep(degparts, u1):
    return pl.pallas_call(
        _tc_prep_body,
        grid=(NB,),
        in_specs=[pl.BlockSpec((2, R, 1), lambda i: (0, i, 0)),
                  pl.BlockSpec((R, D_HID), lambda i: (i, 0))],
        out_specs=[pl.BlockSpec((R, D_HID), lambda i: (i, 0)),
                   pl.BlockSpec((R, D_HID), lambda i: (i, 0))],
        out_shape=[jax.ShapeDtypeStruct((N, D_HID), jnp.float32),
                   jax.ShapeDtypeStruct((N, D_HID), jnp.float32)],
    )(degparts, u1)


def _tc_mid_body(p_ref, y1_ref, dinv_ref, b1_ref, w2_ref, y2_ref):
    h = dinv_ref[...] * (p_ref[0] + p_ref[1] + y1_ref[...]) + b1_ref[...]
    h = jnp.maximum(h, 0.0)
    y2_ref[...] = jnp.dot(h, w2_ref[...],
                          preferred_element_type=jnp.float32) * dinv_ref[...]


def _tc_mid(parts1, y1, dinv16, b1, w2):
    return pl.pallas_call(
        _tc_mid_body,
        grid=(NB,),
        in_specs=[pl.BlockSpec((2, R, D_HID), lambda i: (0, i, 0)),
                  pl.BlockSpec((R, D_HID), lambda i: (i, 0)),
                  pl.BlockSpec((R, D_HID), lambda i: (i, 0)),
                  pl.BlockSpec((1, D_HID), lambda i: (0, 0)),
                  pl.BlockSpec((D_HID, D_HID), lambda i: (0, 0))],
        out_specs=pl.BlockSpec((R, D_HID), lambda i: (i, 0)),
        out_shape=jax.ShapeDtypeStruct((N, D_HID), jnp.float32),
    )(parts1, y1, dinv16, b1, w2)


def _tc_final_body(p_ref, y2_ref, dinv_ref, b2_ref, batch_ref, wl_ref,
                   bl_ref, o_ref, sums, cnts):
    i = pl.program_id(0)

    @pl.when(i == 0)
    def _init():
        sums[...] = jnp.zeros((G, D_HID), jnp.float32)
        cnts[...] = jnp.zeros((G, 1), jnp.float32)

    h2 = dinv_ref[...] * (p_ref[0] + p_ref[1] + y2_ref[...]) + b2_ref[...]
    bid = batch_ref[0]
    seg = jax.lax.broadcasted_iota(jnp.int32, (G, R), 0)
    oh = (seg == jnp.broadcast_to(bid, (G, R))).astype(jnp.float32)
    sums[...] += jnp.dot(oh, h2, preferred_element_type=jnp.float32)
    cnts[...] += jnp.sum(oh, axis=1, keepdims=True)

    @pl.when(i == NB - 1)
    def _fin():
        pooled = sums[...] / jnp.maximum(cnts[...], 1.0)
        logits = jnp.dot(pooled, wl_ref[...],
                         preferred_element_type=jnp.float32) + bl_ref[...]
        m = jnp.max(logits, axis=1, keepdims=True)
        e = jnp.exp(logits - m)
        o_ref[...] = e / jnp.sum(e, axis=1, keepdims=True)


def _tc_final(parts2, y2, dinv16, b2, batch3d, wl, bl):
    return pl.pallas_call(
        _tc_final_body,
        grid=(NB,),
        in_specs=[pl.BlockSpec((2, R, D_HID), lambda i: (0, i, 0)),
                  pl.BlockSpec((R, D_HID), lambda i: (i, 0)),
                  pl.BlockSpec((R, D_HID), lambda i: (i, 0)),
                  pl.BlockSpec((1, D_HID), lambda i: (0, 0)),
                  pl.BlockSpec((1, 1, R), lambda i: (i, 0, 0)),
                  pl.BlockSpec((D_HID, D_OUT), lambda i: (0, 0)),
                  pl.BlockSpec((1, D_OUT), lambda i: (0, 0))],
        out_specs=pl.BlockSpec((G, D_OUT), lambda i: (0, 0)),
        out_shape=jax.ShapeDtypeStruct((G, D_OUT), jnp.float32),
        scratch_shapes=[pltpu.VMEM((G, D_HID), jnp.float32),
                        pltpu.VMEM((G, 1), jnp.float32)],
    )(parts2, y2, dinv16, b2, batch3d, wl, bl)


def kernel(x, edge_index, batch, W1, b1, W2, b2, Wl, bl):
    src = edge_index[0].astype(jnp.int32)
    dst = edge_index[1].astype(jnp.int32)
    pad = jnp.arange(E_PAD - E, dtype=jnp.int32) % C
    src_p = jnp.concatenate([src, pad]).reshape(NCHUNK, C)
    dst_p = jnp.concatenate([dst, N + pad]).reshape(NCHUNK, C)

    z1 = jnp.zeros((NPAD, 1), jnp.float32)
    z16 = jnp.zeros((NPAD, D_HID), jnp.float32)
    ones = jnp.ones((C, 1), jnp.float32)

    degparts = _sc_degree(dst_p, z1, ones)
    u1 = _tc_matmul(x, W1)
    y1, dinv16 = _tc_prep(degparts, u1)
    parts1 = _sc_scatter(y1, src_p, dst_p, z16)
    y2 = _tc_mid(parts1, y1, dinv16, b1.reshape(1, D_HID), W2)
    parts2 = _sc_scatter(y2, src_p, dst_p, z16)
    batch3d = batch.astype(jnp.int32).reshape(NB, 1, R)
    return _tc_final(parts2, y2, dinv16, b2.reshape(1, D_HID), batch3d,
                     Wl, bl.reshape(1, D_OUT))


# speedup vs baseline: 77.1472x; 1.9032x over previous
import functools

import jax
import jax.numpy as jnp
from jax import lax
from jax.experimental import pallas as pl
from jax.experimental.pallas import tpu as pltpu
from jax.experimental.pallas import tpu_sc as plsc

N = 100000
E = 1600000
G = 64
D_IN, D_HID, D_OUT = 32, 16, 5

NPAD = 100352
RPT = NPAD // 16
WROWS = NPAD // 8
NREAL = N // 8

C = 128
NCHUNK = E // C
KB = 10
NBLK = 39

WB = WROWS // 8
RX = NPAD // 16

_MESH = plsc.VectorSubcoreMesh(core_axis_name="c", subcore_axis_name="s")
_SC_PARAMS = pltpu.CompilerParams(use_tc_tiling_on_sc=False)


def _worker(c, s):
    return s * 2 + c


def _chunk0(w):
    return w * 390 + jnp.minimum(w, 20)


@functools.partial(
    pl.kernel,
    out_type=jax.ShapeDtypeStruct((2, NPAD, D_HID), jnp.float32),
    mesh=_MESH,
    scratch_types=[
        pltpu.VMEM((KB, C), jnp.int32),
        pltpu.VMEM((C, D_HID), jnp.float32),
        pltpu.VMEM_SHARED((NPAD, D_HID), jnp.float32),
        pltpu.SemaphoreType.DMA,
    ],
    compiler_params=_SC_PARAMS,
)
def _sc_degree(edges_hbm, z16_hbm, ones_hbm, parts_hbm, dstv, onesv, acc, sem):
    c = lax.axis_index("c")
    s = lax.axis_index("s")
    w = _worker(c, s)
    r0 = s * RPT
    pltpu.sync_copy(z16_hbm.at[pl.ds(r0, RPT)], acc.at[pl.ds(r0, RPT)])
    pltpu.sync_copy(ones_hbm, onesv)
    plsc.subcore_barrier()
    c0 = _chunk0(w)

    def block(b, carry):
        pltpu.sync_copy(edges_hbm.at[1, pl.ds(c0 + b * KB, KB)], dstv)
        ds = [pltpu.async_copy(onesv, acc.at[dstv.at[j]], sem, add=True)
              for j in range(KB)]
        for d in ds:
            d.wait()
        return carry

    lax.fori_loop(0, NBLK, block, 0)

    @pl.when(w < 20)
    def _tail():
        pltpu.sync_copy(edges_hbm.at[1, pl.ds(c0 + NBLK * KB, 1)],
                        dstv.at[pl.ds(0, 1)])
        pltpu.async_copy(onesv, acc.at[dstv.at[0]], sem, add=True).wait()

    plsc.subcore_barrier()
    pltpu.sync_copy(acc.at[pl.ds(r0, RPT)], parts_hbm.at[c].at[pl.ds(r0, RPT)])


@functools.partial(
    pl.kernel,
    out_type=jax.ShapeDtypeStruct((2, NPAD, D_HID), jnp.float32),
    mesh=_MESH,
    scratch_types=[
        pltpu.VMEM((KB, C), jnp.int32),
        pltpu.VMEM((KB, C), jnp.int32),
        pltpu.VMEM((KB, C, D_HID), jnp.float32),
        pltpu.VMEM_SHARED((NPAD, D_HID), jnp.float32),
        pltpu.SemaphoreType.DMA,
        pltpu.SemaphoreType.DMA,
    ],
    compiler_params=_SC_PARAMS,
)
def _sc_scatter(y_hbm, edges_hbm, z16_hbm, parts_hbm,
                srcv, dstv, rows, acc, semg, sems):
    c = lax.axis_index("c")
    s = lax.axis_index("s")
    w = _worker(c, s)
    r0 = s * RPT
    pltpu.sync_copy(z16_hbm.at[pl.ds(r0, RPT)], acc.at[pl.ds(r0, RPT)])
    plsc.subcore_barrier()
    c0 = _chunk0(w)

    def block(b, carry):
        base = c0 + b * KB
        pltpu.sync_copy(edges_hbm.at[0, pl.ds(base, KB)], srcv)
        pltpu.sync_copy(edges_hbm.at[1, pl.ds(base, KB)], dstv)
        gs = [pltpu.async_copy(y_hbm.at[srcv.at[j]], rows.at[j], semg)
              for j in range(KB)]
        for d in gs:
            d.wait()
        ss = [pltpu.async_copy(rows.at[j], acc.at[dstv.at[j]], sems, add=True)
              for j in range(KB)]
        for d in ss:
            d.wait()
        return carry

    lax.fori_loop(0, NBLK, block, 0)

    @pl.when(w < 20)
    def _tail():
        base = c0 + NBLK * KB
        pltpu.sync_copy(edges_hbm.at[0, pl.ds(base, 1)], srcv.at[pl.ds(0, 1)])
        pltpu.sync_copy(edges_hbm.at[1, pl.ds(base, 1)], dstv.at[pl.ds(0, 1)])
        pltpu.async_copy(y_hbm.at[srcv.at[0]], rows.at[0], semg).wait()
        pltpu.async_copy(rows.at[0], acc.at[dstv.at[0]], sems, add=True).wait()

    plsc.subcore_barrier()
    pltpu.sync_copy(acc.at[pl.ds(r0, RPT)], parts_hbm.at[c].at[pl.ds(r0, RPT)])


def _tc_matmul_body(x_ref, w_ref, o_ref):
    o_ref[...] = jnp.dot(x_ref[...], w_ref[...],
                         preferred_element_type=jnp.float32)


def _tc_matmul(x8, w1big):
    return pl.pallas_call(
        _tc_matmul_body,
        grid=(16,),
        in_specs=[pl.BlockSpec((RX // 8, 8 * D_IN), lambda i: (i, 0)),
                  pl.BlockSpec((8 * D_IN, 128), lambda i: (0, 0))],
        out_specs=pl.BlockSpec((RX // 8, 128), lambda i: (i, 0)),
        out_shape=jax.ShapeDtypeStruct((WROWS, 128), jnp.float32),
    )(x8, w1big)


def _tc_prep_body(dp_ref, u_ref, y_ref, dinv_ref):
    i = pl.program_id(0)
    dinv = lax.rsqrt(dp_ref[0] + dp_ref[1] + 1.0)
    wrow = i * WB + jax.lax.broadcasted_iota(jnp.int32, (WB, 128), 0)
    u = jnp.where(wrow < NREAL, u_ref[...], 0.0)
    y_ref[...] = u * dinv
    dinv_ref[...] = dinv


def _tc_prep(degparts_w, u1_w):
    return pl.pallas_call(
        _tc_prep_body,
        grid=(8,),
        in_specs=[pl.BlockSpec((2, WB, 128), lambda i: (0, i, 0)),
                  pl.BlockSpec((WB, 128), lambda i: (i, 0))],
        out_specs=[pl.BlockSpec((WB, 128), lambda i: (i, 0)),
                   pl.BlockSpec((WB, 128), lambda i: (i, 0))],
        out_shape=[jax.ShapeDtypeStruct((WROWS, 128), jnp.float32),
                   jax.ShapeDtypeStruct((WROWS, 128), jnp.float32)],
    )(degparts_w, u1_w)


def _tc_mid_body(p_ref, y1_ref, dinv_ref, b1_ref, w2_ref, y2_ref):
    h = dinv_ref[...] * (p_ref[0] + p_ref[1] + y1_ref[...]) + b1_ref[...]
    h = jnp.maximum(h, 0.0)
    y2_ref[...] = jnp.dot(h, w2_ref[...],
                          preferred_element_type=jnp.float32) * dinv_ref[...]


def _tc_mid(parts1_w, y1_w, dinv_w, b1w, w2big):
    return pl.pallas_call(
        _tc_mid_body,
        grid=(8,),
        in_specs=[pl.BlockSpec((2, WB, 128), lambda i: (0, i, 0)),
                  pl.BlockSpec((WB, 128), lambda i: (i, 0)),
                  pl.BlockSpec((WB, 128), lambda i: (i, 0)),
                  pl.BlockSpec((1, 128), lambda i: (0, 0)),
                  pl.BlockSpec((128, 128), lambda i: (0, 0))],
        out_specs=pl.BlockSpec((WB, 128), lambda i: (i, 0)),
        out_shape=jax.ShapeDtypeStruct((WROWS, 128), jnp.float32),
    )(parts1_w, y1_w, dinv_w, b1w, w2big)


def _tc_mid2_body(p_ref, y2_ref, dinv_ref, b2_ref, h2_ref):
    h2_ref[...] = (dinv_ref[...] * (p_ref[0] + p_ref[1] + y2_ref[...])
                   + b2_ref[...])


def _tc_mid2(parts2_w, y2_w, dinv_w, b2w):
    return pl.pallas_call(
        _tc_mid2_body,
        grid=(8,),
        in_specs=[pl.BlockSpec((2, WB, 128), lambda i: (0, i, 0)),
                  pl.BlockSpec((WB, 128), lambda i: (i, 0)),
                  pl.BlockSpec((WB, 128), lambda i: (i, 0)),
                  pl.BlockSpec((1, 128), lambda i: (0, 0))],
        out_specs=pl.BlockSpec((WB, 128), lambda i: (i, 0)),
        out_shape=jax.ShapeDtypeStruct((WROWS, 128), jnp.float32),
    )(parts2_w, y2_w, dinv_w, b2w)


NPOOL = 80
BCHUNK = NPAD // C
KP = 8
PBLK = 3


@functools.partial(
    pl.kernel,
    out_type=[jax.ShapeDtypeStruct((2, NPOOL, D_HID), jnp.float32),
              jax.ShapeDtypeStruct((2, NPOOL, D_HID), jnp.float32)],
    mesh=_MESH,
    scratch_types=[
        pltpu.VMEM((KP, C), jnp.int32),
        pltpu.VMEM((KP * C, D_HID), jnp.float32),
        pltpu.VMEM((C, D_HID), jnp.float32),
        pltpu.VMEM_SHARED((NPOOL, D_HID), jnp.float32),
        pltpu.VMEM_SHARED((NPOOL, D_HID), jnp.float32),
        pltpu.SemaphoreType.DMA,
        pltpu.SemaphoreType.DMA,
    ],
    compiler_params=_SC_PARAMS,
)
def _sc_pool(h2_hbm, batch_hbm, z16_hbm, ones_hbm, hsum_hbm, cnt_hbm,
             bidx, rows, onesv, hacc, cacc, semh, semc):
    c = lax.axis_index("c")
    s = lax.axis_index("s")
    w = _worker(c, s)
    rp = s * (NPOOL // 16)
    pltpu.sync_copy(z16_hbm.at[pl.ds(rp, NPOOL // 16)],
                    hacc.at[pl.ds(rp, NPOOL // 16)])
    pltpu.sync_copy(z16_hbm.at[pl.ds(NPOOL // 16 * 16 + rp, NPOOL // 16)],
                    cacc.at[pl.ds(rp, NPOOL // 16)])
    pltpu.sync_copy(ones_hbm, onesv)
    plsc.subcore_barrier()
    p0 = w * 24 + jnp.minimum(w, 16)

    def block(b, carry):
        base = p0 + b * KP
        pltpu.sync_copy(batch_hbm.at[pl.ds(base, KP)], bidx)
        pltpu.sync_copy(h2_hbm.at[pl.ds(base * C, KP * C)], rows)
        hs = [pltpu.async_copy(rows.at[pl.ds(j * C, C)],
                               hacc.at[bidx.at[j]], semh, add=True)
              for j in range(KP)]
        cs = [pltpu.async_copy(onesv, cacc.at[bidx.at[j]], semc, add=True)
              for j in range(KP)]
        for d in hs + cs:
            d.wait()
        return carry

    lax.fori_loop(0, PBLK, block, 0)

    @pl.when(w < 16)
    def _tail():
        base = p0 + PBLK * KP
        pltpu.sync_copy(batch_hbm.at[pl.ds(base, 1)], bidx.at[pl.ds(0, 1)])
        pltpu.sync_copy(h2_hbm.at[pl.ds(base * C, C)], rows.at[pl.ds(0, C)])
        pltpu.async_copy(rows.at[pl.ds(0, C)], hacc.at[bidx.at[0]],
                         semh, add=True).wait()
        pltpu.async_copy(onesv, cacc.at[bidx.at[0]], semc, add=True).wait()

    plsc.subcore_barrier()
    pltpu.sync_copy(hacc.at[pl.ds(rp, NPOOL // 16)],
                    hsum_hbm.at[c].at[pl.ds(rp, NPOOL // 16)])
    pltpu.sync_copy(cacc.at[pl.ds(rp, NPOOL // 16)],
                    cnt_hbm.at[c].at[pl.ds(rp, NPOOL // 16)])


def _tc_out_body(hp_ref, cp_ref, wl_ref, bl_ref, o_ref):
    sums = hp_ref[0, :G, :] + hp_ref[1, :G, :]
    cnts = cp_ref[0, :G, 0:1] + cp_ref[1, :G, 0:1]
    pooled = sums / jnp.maximum(cnts, 1.0)
    logits = jnp.dot(pooled, wl_ref[...],
                     preferred_element_type=jnp.float32) + bl_ref[...]
    m = jnp.max(logits, axis=1, keepdims=True)
    e = jnp.exp(logits - m)
    o_ref[...] = e / jnp.sum(e, axis=1, keepdims=True)


def _tc_out(hsum, cnt, wl, blw):
    return pl.pallas_call(
        _tc_out_body,
        in_specs=[pl.BlockSpec((2, NPOOL, D_HID), lambda: (0, 0, 0)),
                  pl.BlockSpec((2, NPOOL, D_HID), lambda: (0, 0, 0)),
                  pl.BlockSpec((D_HID, D_OUT), lambda: (0, 0)),
                  pl.BlockSpec((1, D_OUT), lambda: (0, 0))],
        out_specs=pl.BlockSpec((G, D_OUT), lambda: (0, 0)),
        out_shape=jax.ShapeDtypeStruct((G, D_OUT), jnp.float32),
    )(hsum, cnt, wl, blw)


def kernel(x, edge_index, batch, W1, b1, W2, b2, Wl, bl):
    edges3 = edge_index.astype(jnp.int32).reshape(2, NCHUNK, C)
    z16 = jnp.zeros((NPAD, D_HID), jnp.float32)
    ones16 = jnp.ones((C, D_HID), jnp.float32)

    x8 = x.reshape(NREAL, 8 * D_IN)
    w1big = jnp.kron(jnp.eye(8, dtype=jnp.float32), W1)
    w2big = jnp.kron(jnp.eye(8, dtype=jnp.float32), W2)
    b1w = jnp.tile(b1, 8).reshape(1, 128)
    b2w = jnp.tile(b2, 8).reshape(1, 128)

    degparts = _sc_degree(edges3, z16, ones16)
    u1_w = _tc_matmul(x8, w1big)
    y1_w, dinv_w = _tc_prep(degparts.reshape(2, WROWS, 128), u1_w)
    parts1 = _sc_scatter(y1_w.reshape(NPAD, D_HID), edges3, z16)
    y2_w = _tc_mid(parts1.reshape(2, WROWS, 128), y1_w, dinv_w, b1w, w2big)
    parts2 = _sc_scatter(y2_w.reshape(NPAD, D_HID), edges3, z16)
    h2_w = _tc_mid2(parts2.reshape(2, WROWS, 128), y2_w, dinv_w, b2w)

    batchc = jnp.pad(batch.astype(jnp.int32), (0, NPAD - N),
                     constant_values=G).reshape(BCHUNK, C)
    hsum, cnt = _sc_pool(h2_w.reshape(NPAD, D_HID), batchc, z16, ones16)
    return _tc_out(hsum, cnt, Wl, bl.reshape(1, D_OUT))


# speedup vs baseline: 92.0596x; 1.1933x over previous
import functools

import jax
import jax.numpy as jnp
from jax import lax
from jax.experimental import pallas as pl
from jax.experimental.pallas import tpu as pltpu
from jax.experimental.pallas import tpu_sc as plsc

N = 100000
E = 1600000
G = 64
D_IN, D_HID, D_OUT = 32, 16, 5

NPAD = 100352
RPT = NPAD // 16
WROWS = NPAD // 8
NREAL = N // 8

C = 128
NCHUNK = E // C
KB = 10
NBLK = 39

WB = WROWS // 8
RX = NPAD // 16

_MESH = plsc.VectorSubcoreMesh(core_axis_name="c", subcore_axis_name="s")
_SC_PARAMS = pltpu.CompilerParams(use_tc_tiling_on_sc=False)


def _worker(c, s):
    return s * 2 + c


def _chunk0(w):
    return w * 390 + jnp.minimum(w, 20)


def _fill_rows(ref, nrows, value):
    v = jnp.full((D_HID,), value, jnp.float32)

    def body(i, carry):
        ref[i] = v
        return carry

    lax.fori_loop(0, nrows, body, 0)


@functools.partial(
    pl.kernel,
    out_type=jax.ShapeDtypeStruct((2, NPAD, D_HID), jnp.float32),
    mesh=_MESH,
    scratch_types=[
        pltpu.VMEM((KB, C), jnp.int32),
        pltpu.VMEM((C, D_HID), jnp.float32),
        pltpu.VMEM((512, D_HID), jnp.float32),
        pltpu.VMEM_SHARED((NPAD, D_HID), jnp.float32),
        pltpu.SemaphoreType.DMA,
    ],
    compiler_params=_SC_PARAMS,
)
def _sc_degree(edges_hbm, parts_hbm, dstv, onesv, zbuf, acc, sem):
    c = lax.axis_index("c")
    s = lax.axis_index("s")
    w = _worker(c, s)
    r0 = s * RPT
    _fill_rows(zbuf, 512, 0.0)
    _fill_rows(onesv, C, 1.0)

    def zcp(t, carry):
        pltpu.sync_copy(zbuf, acc.at[pl.ds(r0 + t * 512, 512)])
        return carry

    lax.fori_loop(0, 12, zcp, 0)
    pltpu.sync_copy(zbuf.at[pl.ds(0, 128)],
                    acc.at[pl.ds(r0 + 6144, 128)])
    plsc.subcore_barrier()
    c0 = _chunk0(w)

    def block(b, carry):
        pltpu.sync_copy(edges_hbm.at[1, pl.ds(c0 + b * KB, KB)], dstv)
        ds = [pltpu.async_copy(onesv, acc.at[dstv.at[j]], sem, add=True)
              for j in range(KB)]
        for d in ds:
            d.wait()
        return carry

    lax.fori_loop(0, NBLK, block, 0)

    @pl.when(w < 20)
    def _tail():
        pltpu.sync_copy(edges_hbm.at[1, pl.ds(c0 + NBLK * KB, 1)],
                        dstv.at[pl.ds(0, 1)])
        pltpu.async_copy(onesv, acc.at[dstv.at[0]], sem, add=True).wait()

    plsc.subcore_barrier()
    pltpu.sync_copy(acc.at[pl.ds(r0, RPT)], parts_hbm.at[c].at[pl.ds(r0, RPT)])


@functools.partial(
    pl.kernel,
    out_type=jax.ShapeDtypeStruct((2, NPAD, D_HID), jnp.float32),
    mesh=_MESH,
    scratch_types=[
        pltpu.VMEM((KB, C), jnp.int32),
        pltpu.VMEM((KB, C), jnp.int32),
        pltpu.VMEM((KB, C, D_HID), jnp.float32),
        pltpu.VMEM_SHARED((NPAD, D_HID), jnp.float32),
        pltpu.SemaphoreType.DMA,
        pltpu.SemaphoreType.DMA,
        pltpu.SemaphoreType.DMA,
        pltpu.SemaphoreType.DMA,
        pltpu.SemaphoreType.DMA,
        pltpu.SemaphoreType.DMA,
    ],
    compiler_params=_SC_PARAMS,
)
def _sc_scatter(y_hbm, edges_hbm, z16_hbm, parts_hbm,
                srcv, dstv, rows, acc, semg, sems, semg2, sems2, semi, semi2):
    c = lax.axis_index("c")
    s = lax.axis_index("s")
    w = _worker(c, s)
    r0 = s * RPT
    pltpu.sync_copy(z16_hbm.at[pl.ds(r0, RPT)], acc.at[pl.ds(r0, RPT)])
    plsc.subcore_barrier()
    c0 = _chunk0(w)
    H = KB // 2

    def block(b, carry):
        base = c0 + b * KB
        di = pltpu.async_copy(edges_hbm.at[0, pl.ds(base, KB)], srcv, semi)
        dd = pltpu.async_copy(edges_hbm.at[1, pl.ds(base, KB)], dstv, semi2)
        di.wait()
        ga = [pltpu.async_copy(y_hbm.at[srcv.at[j]], rows.at[j], semg)
              for j in range(H)]
        gb = [pltpu.async_copy(y_hbm.at[srcv.at[j]], rows.at[j], semg2)
              for j in range(H, KB)]
        dd.wait()
        for d in ga:
            d.wait()
        sa = [pltpu.async_copy(rows.at[j], acc.at[dstv.at[j]], sems, add=True)
              for j in range(H)]
        for d in gb:
            d.wait()
        sb = [pltpu.async_copy(rows.at[j], acc.at[dstv.at[j]], sems2, add=True)
              for j in range(H, KB)]
        for d in sa + sb:
            d.wait()
        return carry

    lax.fori_loop(0, NBLK, block, 0)

    @pl.when(w < 20)
    def _tail():
        base = c0 + NBLK * KB
        pltpu.sync_copy(edges_hbm.at[0, pl.ds(base, 1)], srcv.at[pl.ds(0, 1)])
        pltpu.sync_copy(edges_hbm.at[1, pl.ds(base, 1)], dstv.at[pl.ds(0, 1)])
        pltpu.async_copy(y_hbm.at[srcv.at[0]], rows.at[0], semg).wait()
        pltpu.async_copy(rows.at[0], acc.at[dstv.at[0]], sems, add=True).wait()

    plsc.subcore_barrier()
    pltpu.sync_copy(acc.at[pl.ds(r0, RPT)], parts_hbm.at[c].at[pl.ds(r0, RPT)])


def _tc_matmul_body(x_ref, w_ref, o_ref):
    o_ref[...] = jnp.dot(x_ref[...], w_ref[...],
                         preferred_element_type=jnp.float32)


def _tc_matmul(x8, w1big):
    return pl.pallas_call(
        _tc_matmul_body,
        grid=(16,),
        in_specs=[pl.BlockSpec((RX // 8, 8 * D_IN), lambda i: (i, 0)),
                  pl.BlockSpec((8 * D_IN, 128), lambda i: (0, 0))],
        out_specs=pl.BlockSpec((RX // 8, 128), lambda i: (i, 0)),
        out_shape=jax.ShapeDtypeStruct((WROWS, 128), jnp.float32),
    )(x8, w1big)


def _tc_prep_body(dp_ref, u_ref, y_ref, dinv_ref):
    i = pl.program_id(0)
    dinv = lax.rsqrt(dp_ref[0] + dp_ref[1] + 1.0)
    wrow = i * WB + jax.lax.broadcasted_iota(jnp.int32, (WB, 128), 0)
    u = jnp.where(wrow < NREAL, u_ref[...], 0.0)
    y_ref[...] = u * dinv
    dinv_ref[...] = dinv


def _tc_prep(degparts_w, u1_w):
    return pl.pallas_call(
        _tc_prep_body,
        grid=(8,),
        in_specs=[pl.BlockSpec((2, WB, 128), lambda i: (0, i, 0)),
                  pl.BlockSpec((WB, 128), lambda i: (i, 0))],
        out_specs=[pl.BlockSpec((WB, 128), lambda i: (i, 0)),
                   pl.BlockSpec((WB, 128), lambda i: (i, 0))],
        out_shape=[jax.ShapeDtypeStruct((WROWS, 128), jnp.float32),
                   jax.ShapeDtypeStruct((WROWS, 128), jnp.float32)],
    )(degparts_w, u1_w)


def _tc_mid_body(p_ref, y1_ref, dinv_ref, b1_ref, w2_ref, y2_ref):
    h = dinv_ref[...] * (p_ref[0] + p_ref[1] + y1_ref[...]) + b1_ref[...]
    h = jnp.maximum(h, 0.0)
    y2_ref[...] = jnp.dot(h, w2_ref[...],
                          preferred_element_type=jnp.float32) * dinv_ref[...]


def _tc_mid(parts1_w, y1_w, dinv_w, b1w, w2big):
    return pl.pallas_call(
        _tc_mid_body,
        grid=(8,),
        in_specs=[pl.BlockSpec((2, WB, 128), lambda i: (0, i, 0)),
                  pl.BlockSpec((WB, 128), lambda i: (i, 0)),
                  pl.BlockSpec((WB, 128), lambda i: (i, 0)),
                  pl.BlockSpec((1, 128), lambda i: (0, 0)),
                  pl.BlockSpec((128, 128), lambda i: (0, 0))],
        out_specs=pl.BlockSpec((WB, 128), lambda i: (i, 0)),
        out_shape=jax.ShapeDtypeStruct((WROWS, 128), jnp.float32),
    )(parts1_w, y1_w, dinv_w, b1w, w2big)


def _tc_mid2_body(p_ref, y2_ref, dinv_ref, b2_ref, h2_ref):
    h2_ref[...] = (dinv_ref[...] * (p_ref[0] + p_ref[1] + y2_ref[...])
                   + b2_ref[...])


def _tc_mid2(parts2_w, y2_w, dinv_w, b2w):
    return pl.pallas_call(
        _tc_mid2_body,
        grid=(8,),
        in_specs=[pl.BlockSpec((2, WB, 128), lambda i: (0, i, 0)),
                  pl.BlockSpec((WB, 128), lambda i: (i, 0)),
                  pl.BlockSpec((WB, 128), lambda i: (i, 0)),
                  pl.BlockSpec((1, 128), lambda i: (0, 0))],
        out_specs=pl.BlockSpec((WB, 128), lambda i: (i, 0)),
        out_shape=jax.ShapeDtypeStruct((WROWS, 128), jnp.float32),
    )(parts2_w, y2_w, dinv_w, b2w)


NPOOL = 80
BCHUNK = NPAD // C
KP = 8
PBLK = 3


@functools.partial(
    pl.kernel,
    out_type=[jax.ShapeDtypeStruct((2, NPOOL, D_HID), jnp.float32),
              jax.ShapeDtypeStruct((2, NPOOL, D_HID), jnp.float32)],
    mesh=_MESH,
    scratch_types=[
        pltpu.VMEM((KP, C), jnp.int32),
        pltpu.VMEM((KP * C, D_HID), jnp.float32),
        pltpu.VMEM((C, D_HID), jnp.float32),
        pltpu.VMEM_SHARED((NPOOL, D_HID), jnp.float32),
        pltpu.VMEM_SHARED((NPOOL, D_HID), jnp.float32),
        pltpu.SemaphoreType.DMA,
        pltpu.SemaphoreType.DMA,
    ],
    compiler_params=_SC_PARAMS,
)
def _sc_pool(h2_hbm, batch_hbm, hsum_hbm, cnt_hbm,
             bidx, rows, onesv, hacc, cacc, semh, semc):
    c = lax.axis_index("c")
    s = lax.axis_index("s")
    w = _worker(c, s)
    rp = s * (NPOOL // 16)
    _fill_rows(onesv, NPOOL // 16, 0.0)
    pltpu.sync_copy(onesv.at[pl.ds(0, NPOOL // 16)],
                    hacc.at[pl.ds(rp, NPOOL // 16)])
    pltpu.sync_copy(onesv.at[pl.ds(0, NPOOL // 16)],
                    cacc.at[pl.ds(rp, NPOOL // 16)])
    _fill_rows(onesv, C, 1.0)
    plsc.subcore_barrier()
    p0 = w * 24 + jnp.minimum(w, 16)

    def block(b, carry):
        base = p0 + b * KP
        pltpu.sync_copy(batch_hbm.at[pl.ds(base, KP)], bidx)
        pltpu.sync_copy(h2_hbm.at[pl.ds(base * C, KP * C)], rows)
        hs = [pltpu.async_copy(rows.at[pl.ds(j * C, C)],
                               hacc.at[bidx.at[j]], semh, add=True)
              for j in range(KP)]
        cs = [pltpu.async_copy(onesv, cacc.at[bidx.at[j]], semc, add=True)
              for j in range(KP)]
        for d in hs + cs:
            d.wait()
        return carry

    lax.fori_loop(0, PBLK, block, 0)

    @pl.when(w < 16)
    def _tail():
        base = p0 + PBLK * KP
        pltpu.sync_copy(batch_hbm.at[pl.ds(base, 1)], bidx.at[pl.ds(0, 1)])
        pltpu.sync_copy(h2_hbm.at[pl.ds(base * C, C)], rows.at[pl.ds(0, C)])
        pltpu.async_copy(rows.at[pl.ds(0, C)], hacc.at[bidx.at[0]],
                         semh, add=True).wait()
        pltpu.async_copy(onesv, cacc.at[bidx.at[0]], semc, add=True).wait()

    plsc.subcore_barrier()
    pltpu.sync_copy(hacc.at[pl.ds(rp, NPOOL // 16)],
                    hsum_hbm.at[c].at[pl.ds(rp, NPOOL // 16)])
    pltpu.sync_copy(cacc.at[pl.ds(rp, NPOOL // 16)],
                    cnt_hbm.at[c].at[pl.ds(rp, NPOOL // 16)])


def _tc_out_body(hp_ref, cp_ref, wl_ref, bl_ref, o_ref):
    sums = hp_ref[0, :G, :] + hp_ref[1, :G, :]
    cnts = cp_ref[0, :G, 0:1] + cp_ref[1, :G, 0:1]
    pooled = sums / jnp.maximum(cnts, 1.0)
    logits = jnp.dot(pooled, wl_ref[...],
                     preferred_element_type=jnp.float32) + bl_ref[...]
    m = jnp.max(logits, axis=1, keepdims=True)
    e = jnp.exp(logits - m)
    o_ref[...] = e / jnp.sum(e, axis=1, keepdims=True)


def _tc_out(hsum, cnt, wl, blw):
    return pl.pallas_call(
        _tc_out_body,
        in_specs=[pl.BlockSpec((2, NPOOL, D_HID), lambda: (0, 0, 0)),
                  pl.BlockSpec((2, NPOOL, D_HID), lambda: (0, 0, 0)),
                  pl.BlockSpec((D_HID, D_OUT), lambda: (0, 0)),
                  pl.BlockSpec((1, D_OUT), lambda: (0, 0))],
        out_specs=pl.BlockSpec((G, D_OUT), lambda: (0, 0)),
        out_shape=jax.ShapeDtypeStruct((G, D_OUT), jnp.float32),
    )(hsum, cnt, wl, blw)


def kernel(x, edge_index, batch, W1, b1, W2, b2, Wl, bl):
    edges3 = edge_index.astype(jnp.int32).reshape(2, NCHUNK, C)
    z16 = jnp.zeros((NPAD, D_HID), jnp.float32)

    x8 = x.reshape(NREAL, 8 * D_IN)
    w1big = jnp.kron(jnp.eye(8, dtype=jnp.float32), W1)
    w2big = jnp.kron(jnp.eye(8, dtype=jnp.float32), W2)
    b1w = jnp.tile(b1, 8).reshape(1, 128)
    b2w = jnp.tile(b2, 8).reshape(1, 128)

    degparts = _sc_degree(edges3)
    u1_w = _tc_matmul(x8, w1big)
    y1_w, dinv_w = _tc_prep(degparts.reshape(2, WROWS, 128), u1_w)
    parts1 = _sc_scatter(y1_w.reshape(NPAD, D_HID), edges3, z16)
    y2_w = _tc_mid(parts1.reshape(2, WROWS, 128), y1_w, dinv_w, b1w, w2big)
    parts2 = _sc_scatter(y2_w.reshape(NPAD, D_HID), edges3, z16)
    h2_w = _tc_mid2(parts2.reshape(2, WROWS, 128), y2_w, dinv_w, b2w)

    batchc = jnp.pad(batch.astype(jnp.int32), (0, NPAD - N),
                     constant_values=G).reshape(BCHUNK, C)
    hsum, cnt = _sc_pool(h2_w.reshape(NPAD, D_HID), batchc)
    return _tc_out(hsum, cnt, Wl, bl.reshape(1, D_OUT))


# speedup vs baseline: 97.1788x; 1.0556x over previous
import functools

import jax
import jax.numpy as jnp
from jax import lax
from jax.experimental import pallas as pl
from jax.experimental.pallas import tpu as pltpu
from jax.experimental.pallas import tpu_sc as plsc

N = 100000
E = 1600000
G = 64
D_IN, D_HID, D_OUT = 32, 16, 5

NPAD = 100352
RPT = NPAD // 16
WROWS = NPAD // 8
NREAL = N // 8

C = 128
NCHUNK = E // C
KB = 10
NBLK = 39

WB = WROWS // 8
RX = NPAD // 16

_MESH = plsc.VectorSubcoreMesh(core_axis_name="c", subcore_axis_name="s")
_SC_PARAMS = pltpu.CompilerParams(use_tc_tiling_on_sc=False)


def _worker(c, s):
    return s * 2 + c


def _chunk0(w):
    return w * 390 + jnp.minimum(w, 20)


def _fill_rows(ref, nrows, value):
    v = jnp.full((D_HID,), value, jnp.float32)

    def body(i, carry):
        ref[i] = v
        return carry

    lax.fori_loop(0, nrows, body, 0)


@functools.partial(
    pl.kernel,
    out_type=jax.ShapeDtypeStruct((2, NPAD, D_HID), jnp.float32),
    mesh=_MESH,
    scratch_types=[
        pltpu.VMEM((KB, C), jnp.int32),
        pltpu.VMEM((C, D_HID), jnp.float32),
        pltpu.VMEM((512, D_HID), jnp.float32),
        pltpu.VMEM_SHARED((NPAD, D_HID), jnp.float32),
        pltpu.SemaphoreType.DMA,
    ],
    compiler_params=_SC_PARAMS,
)
def _sc_degree(edges_hbm, parts_hbm, dstv, onesv, zbuf, acc, sem):
    c = lax.axis_index("c")
    s = lax.axis_index("s")
    w = _worker(c, s)
    r0 = s * RPT
    _fill_rows(zbuf, 512, 0.0)
    _fill_rows(onesv, C, 1.0)

    def zcp(t, carry):
        pltpu.sync_copy(zbuf, acc.at[pl.ds(r0 + t * 512, 512)])
        return carry

    lax.fori_loop(0, 12, zcp, 0)
    pltpu.sync_copy(zbuf.at[pl.ds(0, 128)],
                    acc.at[pl.ds(r0 + 6144, 128)])
    plsc.subcore_barrier()
    c0 = _chunk0(w)

    def block(b, carry):
        pltpu.sync_copy(edges_hbm.at[1, pl.ds(c0 + b * KB, KB)], dstv)
        ds = [pltpu.async_copy(onesv, acc.at[dstv.at[j]], sem, add=True)
              for j in range(KB)]
        for d in ds:
            d.wait()
        return carry

    lax.fori_loop(0, NBLK, block, 0)

    @pl.when(w < 20)
    def _tail():
        pltpu.sync_copy(edges_hbm.at[1, pl.ds(c0 + NBLK * KB, 1)],
                        dstv.at[pl.ds(0, 1)])
        pltpu.async_copy(onesv, acc.at[dstv.at[0]], sem, add=True).wait()

    plsc.subcore_barrier()
    pltpu.sync_copy(acc.at[pl.ds(r0, RPT)], parts_hbm.at[c].at[pl.ds(r0, RPT)])


@functools.partial(
    pl.kernel,
    out_type=jax.ShapeDtypeStruct((2, NPAD, D_HID), jnp.float32),
    mesh=_MESH,
    scratch_types=[
        pltpu.VMEM((KB, C), jnp.int32),
        pltpu.VMEM((KB, C), jnp.int32),
        pltpu.VMEM((KB, C, D_HID), jnp.float32),
        pltpu.VMEM_SHARED((NPAD, D_HID), jnp.float32),
    ] + [pltpu.SemaphoreType.DMA] * 12,
    compiler_params=_SC_PARAMS,
)
def _sc_scatter(y_hbm, edges_hbm, z16_hbm, parts_hbm,
                srcv, dstv, rows, acc, *sems):
    semg = sems[0:5]
    semsc = sems[5:10]
    semi, semi2 = sems[10], sems[11]
    c = lax.axis_index("c")
    s = lax.axis_index("s")
    w = _worker(c, s)
    r0 = s * RPT
    pltpu.sync_copy(z16_hbm.at[pl.ds(r0, RPT)], acc.at[pl.ds(r0, RPT)])
    plsc.subcore_barrier()
    c0 = _chunk0(w)
    WAVES = [(2 * t, 2 * t + 1) for t in range(KB // 2)]

    def block(b, carry):
        base = c0 + b * KB
        di = pltpu.async_copy(edges_hbm.at[0, pl.ds(base, KB)], srcv, semi)
        dd = pltpu.async_copy(edges_hbm.at[1, pl.ds(base, KB)], dstv, semi2)
        di.wait()
        gs = [[pltpu.async_copy(y_hbm.at[srcv.at[j]], rows.at[j], semg[t])
               for j in wave] for t, wave in enumerate(WAVES)]
        dd.wait()
        ss = []
        for t, wave in enumerate(WAVES):
            for d in gs[t]:
                d.wait()
            ss += [pltpu.async_copy(rows.at[j], acc.at[dstv.at[j]],
                                    semsc[t], add=True) for j in wave]
        for d in ss:
            d.wait()
        return carry

    lax.fori_loop(0, NBLK, block, 0)

    @pl.when(w < 20)
    def _tail():
        base = c0 + NBLK * KB
        pltpu.sync_copy(edges_hbm.at[0, pl.ds(base, 1)], srcv.at[pl.ds(0, 1)])
        pltpu.sync_copy(edges_hbm.at[1, pl.ds(base, 1)], dstv.at[pl.ds(0, 1)])
        pltpu.async_copy(y_hbm.at[srcv.at[0]], rows.at[0], semg[0]).wait()
        pltpu.async_copy(rows.at[0], acc.at[dstv.at[0]],
                         semsc[0], add=True).wait()

    plsc.subcore_barrier()
    pltpu.sync_copy(acc.at[pl.ds(r0, RPT)], parts_hbm.at[c].at[pl.ds(r0, RPT)])


def _tc_matmul_body(x_ref, w_ref, o_ref):
    o_ref[...] = jnp.dot(x_ref[...], w_ref[...],
                         preferred_element_type=jnp.float32)


def _tc_matmul(x8, w1big):
    return pl.pallas_call(
        _tc_matmul_body,
        grid=(16,),
        in_specs=[pl.BlockSpec((RX // 8, 8 * D_IN), lambda i: (i, 0)),
                  pl.BlockSpec((8 * D_IN, 128), lambda i: (0, 0))],
        out_specs=pl.BlockSpec((RX // 8, 128), lambda i: (i, 0)),
        out_shape=jax.ShapeDtypeStruct((WROWS, 128), jnp.float32),
    )(x8, w1big)


def _tc_prep_body(dp_ref, u_ref, y_ref, dinv_ref):
    i = pl.program_id(0)
    dinv = lax.rsqrt(dp_ref[0] + dp_ref[1] + 1.0)
    wrow = i * WB + jax.lax.broadcasted_iota(jnp.int32, (WB, 128), 0)
    u = jnp.where(wrow < NREAL, u_ref[...], 0.0)
    y_ref[...] = u * dinv
    dinv_ref[...] = dinv


def _tc_prep(degparts_w, u1_w):
    return pl.pallas_call(
        _tc_prep_body,
        grid=(8,),
        in_specs=[pl.BlockSpec((2, WB, 128), lambda i: (0, i, 0)),
                  pl.BlockSpec((WB, 128), lambda i: (i, 0))],
        out_specs=[pl.BlockSpec((WB, 128), lambda i: (i, 0)),
                   pl.BlockSpec((WB, 128), lambda i: (i, 0))],
        out_shape=[jax.ShapeDtypeStruct((WROWS, 128), jnp.float32),
                   jax.ShapeDtypeStruct((WROWS, 128), jnp.float32)],
    )(degparts_w, u1_w)


def _tc_mid_body(p_ref, y1_ref, dinv_ref, b1_ref, w2_ref, y2_ref):
    h = dinv_ref[...] * (p_ref[0] + p_ref[1] + y1_ref[...]) + b1_ref[...]
    h = jnp.maximum(h, 0.0)
    y2_ref[...] = jnp.dot(h, w2_ref[...],
                          preferred_element_type=jnp.float32) * dinv_ref[...]


def _tc_mid(parts1_w, y1_w, dinv_w, b1w, w2big):
    return pl.pallas_call(
        _tc_mid_body,
        grid=(8,),
        in_specs=[pl.BlockSpec((2, WB, 128), lambda i: (0, i, 0)),
                  pl.BlockSpec((WB, 128), lambda i: (i, 0)),
                  pl.BlockSpec((WB, 128), lambda i: (i, 0)),
                  pl.BlockSpec((1, 128), lambda i: (0, 0)),
                  pl.BlockSpec((128, 128), lambda i: (0, 0))],
        out_specs=pl.BlockSpec((WB, 128), lambda i: (i, 0)),
        out_shape=jax.ShapeDtypeStruct((WROWS, 128), jnp.float32),
    )(parts1_w, y1_w, dinv_w, b1w, w2big)


def _tc_mid2_body(p_ref, y2_ref, dinv_ref, b2_ref, h2_ref):
    h2_ref[...] = (dinv_ref[...] * (p_ref[0] + p_ref[1] + y2_ref[...])
                   + b2_ref[...])


def _tc_mid2(parts2_w, y2_w, dinv_w, b2w):
    return pl.pallas_call(
        _tc_mid2_body,
        grid=(8,),
        in_specs=[pl.BlockSpec((2, WB, 128), lambda i: (0, i, 0)),
                  pl.BlockSpec((WB, 128), lambda i: (i, 0)),
                  pl.BlockSpec((WB, 128), lambda i: (i, 0)),
                  pl.BlockSpec((1, 128), lambda i: (0, 0))],
        out_specs=pl.BlockSpec((WB, 128), lambda i: (i, 0)),
        out_shape=jax.ShapeDtypeStruct((WROWS, 128), jnp.float32),
    )(parts2_w, y2_w, dinv_w, b2w)


NPOOL = 80
BCHUNK = NPAD // C
KP = 8
PBLK = 3


@functools.partial(
    pl.kernel,
    out_type=[jax.ShapeDtypeStruct((2, NPOOL, D_HID), jnp.float32),
              jax.ShapeDtypeStruct((2, NPOOL, D_HID), jnp.float32)],
    mesh=_MESH,
    scratch_types=[
        pltpu.VMEM((KP, C), jnp.int32),
        pltpu.VMEM((KP * C, D_HID), jnp.float32),
        pltpu.VMEM((C, D_HID), jnp.float32),
        pltpu.VMEM_SHARED((NPOOL, D_HID), jnp.float32),
        pltpu.VMEM_SHARED((NPOOL, D_HID), jnp.float32),
        pltpu.SemaphoreType.DMA,
        pltpu.SemaphoreType.DMA,
    ],
    compiler_params=_SC_PARAMS,
)
def _sc_pool(h2_hbm, batch_hbm, hsum_hbm, cnt_hbm,
             bidx, rows, onesv, hacc, cacc, semh, semc):
    c = lax.axis_index("c")
    s = lax.axis_index("s")
    w = _worker(c, s)
    rp = s * (NPOOL // 16)
    _fill_rows(onesv, NPOOL // 16, 0.0)
    pltpu.sync_copy(onesv.at[pl.ds(0, NPOOL // 16)],
                    hacc.at[pl.ds(rp, NPOOL // 16)])
    pltpu.sync_copy(onesv.at[pl.ds(0, NPOOL // 16)],
                    cacc.at[pl.ds(rp, NPOOL // 16)])
    _fill_rows(onesv, C, 1.0)
    plsc.subcore_barrier()
    p0 = w * 24 + jnp.minimum(w, 16)

    def block(b, carry):
        base = p0 + b * KP
        pltpu.sync_copy(batch_hbm.at[pl.ds(base, KP)], bidx)
        pltpu.sync_copy(h2_hbm.at[pl.ds(base * C, KP * C)], rows)
        hs = [pltpu.async_copy(rows.at[pl.ds(j * C, C)],
                               hacc.at[bidx.at[j]], semh, add=True)
              for j in range(KP)]
        cs = [pltpu.async_copy(onesv, cacc.at[bidx.at[j]], semc, add=True)
              for j in range(KP)]
        for d in hs + cs:
            d.wait()
        return carry

    lax.fori_loop(0, PBLK, block, 0)

    @pl.when(w < 16)
    def _tail():
        base = p0 + PBLK * KP
        pltpu.sync_copy(batch_hbm.at[pl.ds(base, 1)], bidx.at[pl.ds(0, 1)])
        pltpu.sync_copy(h2_hbm.at[pl.ds(base * C, C)], rows.at[pl.ds(0, C)])
        pltpu.async_copy(rows.at[pl.ds(0, C)], hacc.at[bidx.at[0]],
                         semh, add=True).wait()
        pltpu.async_copy(onesv, cacc.at[bidx.at[0]], semc, add=True).wait()

    plsc.subcore_barrier()
    pltpu.sync_copy(hacc.at[pl.ds(rp, NPOOL // 16)],
                    hsum_hbm.at[c].at[pl.ds(rp, NPOOL // 16)])
    pltpu.sync_copy(cacc.at[pl.ds(rp, NPOOL // 16)],
                    cnt_hbm.at[c].at[pl.ds(rp, NPOOL // 16)])


def _tc_out_body(hp_ref, cp_ref, wl_ref, bl_ref, o_ref):
    sums = hp_ref[0, :G, :] + hp_ref[1, :G, :]
    cnts = cp_ref[0, :G, 0:1] + cp_ref[1, :G, 0:1]
    pooled = sums / jnp.maximum(cnts, 1.0)
    logits = jnp.dot(pooled, wl_ref[...],
                     preferred_element_type=jnp.float32) + bl_ref[...]
    m = jnp.max(logits, axis=1, keepdims=True)
    e = jnp.exp(logits - m)
    o_ref[...] = e / jnp.sum(e, axis=1, keepdims=True)


def _tc_out(hsum, cnt, wl, blw):
    return pl.pallas_call(
        _tc_out_body,
        in_specs=[pl.BlockSpec((2, NPOOL, D_HID), lambda: (0, 0, 0)),
                  pl.BlockSpec((2, NPOOL, D_HID), lambda: (0, 0, 0)),
                  pl.BlockSpec((D_HID, D_OUT), lambda: (0, 0)),
                  pl.BlockSpec((1, D_OUT), lambda: (0, 0))],
        out_specs=pl.BlockSpec((G, D_OUT), lambda: (0, 0)),
        out_shape=jax.ShapeDtypeStruct((G, D_OUT), jnp.float32),
    )(hsum, cnt, wl, blw)


def kernel(x, edge_index, batch, W1, b1, W2, b2, Wl, bl):
    edges3 = edge_index.astype(jnp.int32).reshape(2, NCHUNK, C)
    z16 = jnp.zeros((NPAD, D_HID), jnp.float32)

    x8 = x.reshape(NREAL, 8 * D_IN)
    w1big = jnp.kron(jnp.eye(8, dtype=jnp.float32), W1)
    w2big = jnp.kron(jnp.eye(8, dtype=jnp.float32), W2)
    b1w = jnp.tile(b1, 8).reshape(1, 128)
    b2w = jnp.tile(b2, 8).reshape(1, 128)

    degparts = _sc_degree(edges3)
    u1_w = _tc_matmul(x8, w1big)
    y1_w, dinv_w = _tc_prep(degparts.reshape(2, WROWS, 128), u1_w)
    parts1 = _sc_scatter(y1_w.reshape(NPAD, D_HID), edges3, z16)
    y2_w = _tc_mid(parts1.reshape(2, WROWS, 128), y1_w, dinv_w, b1w, w2big)
    parts2 = _sc_scatter(y2_w.reshape(NPAD, D_HID), edges3, z16)
    h2_w = _tc_mid2(parts2.reshape(2, WROWS, 128), y2_w, dinv_w, b2w)

    batchc = jnp.pad(batch.astype(jnp.int32), (0, NPAD - N),
                     constant_values=G).reshape(BCHUNK, C)
    hsum, cnt = _sc_pool(h2_w.reshape(NPAD, D_HID), batchc)
    return _tc_out(hsum, cnt, Wl, bl.reshape(1, D_OUT))
